# Initial kernel scaffold; baseline (speedup 1.0000x reference)
import jax
import jax.numpy as jnp
from jax.experimental import pallas as pl


def kernel(x, edge_index, W1, b1, W2, b2):
    raise NotImplementedError("write your pallas kernel here")


# speedup vs baseline: 7.7541x; 7.7541x over previous
import functools

import jax
import jax.numpy as jnp
from jax import lax
from jax.experimental import pallas as pl
from jax.experimental.pallas import tpu as pltpu
from jax.experimental.pallas import tpu_sc as plsc

_N = 10000
_E = 160000
_F = 256
_H = 512
_C = 128

_NS = 16
_NC = 2
_CH = 128
_E_PAD = 163840
_ACC = 10240
_ZR = _ACC // _NS
_OFF = 11000
_DUMMY = _N

_f32 = jnp.float32
_i32 = jnp.int32

_mesh = plsc.VectorSubcoreMesh(core_axis_name="c", subcore_axis_name="s")


@functools.partial(
    pl.kernel,
    out_type=jax.ShapeDtypeStruct((_OFF + _ACC, 128), _f32),
    mesh=_mesh,
    scratch_types=[
        pltpu.VMEM((_CH, 128), _f32),
        pltpu.VMEM((_CH,), _i32),
        pltpu.VMEM_SHARED((_ACC, 128), _f32),
    ],
)
def _deg_k(colp, ones_in, zer, pp, ones_v, idxc, acc):
    c = lax.axis_index("c")
    s = lax.axis_index("s")
    pltpu.sync_copy(zer, acc.at[pl.ds(s * _ZR, _ZR)])
    pltpu.sync_copy(ones_in, ones_v)
    plsc.subcore_barrier()
    base = c * (_E_PAD // 2) + s * (_E_PAD // (2 * _NS))

    def chunk(k, carry):
        pltpu.sync_copy(colp.at[pl.ds(base + k * _CH, _CH)], idxc)
        pltpu.sync_copy(ones_v, acc.at[idxc], add=True)
        return carry

    lax.fori_loop(0, _E_PAD // (2 * _NS) // _CH, chunk, 0)
    plsc.subcore_barrier()
    pltpu.sync_copy(acc.at[pl.ds(s * _ZR, _ZR)],
                    pp.at[pl.ds(c * _OFF + s * _ZR, _ZR)])


@functools.partial(
    pl.kernel,
    out_type=jax.ShapeDtypeStruct((_OFF + _ACC, 128), _f32),
    mesh=_mesh,
    scratch_types=[
        pltpu.VMEM((_CH,), _i32),
        pltpu.VMEM((_CH,), _i32),
        pltpu.VMEM((_CH, 128), _f32),
        pltpu.VMEM_SHARED((_ACC, 128), _f32),
        pltpu.SemaphoreType.DMA,
    ],
)
def _agg1_k(ya, yb, rowp, colp, zer, out, idxr, idxc, rows, acc, sem):
    c = lax.axis_index("c")
    s = lax.axis_index("s")
    pltpu.sync_copy(zer, acc.at[pl.ds(s * _ZR, _ZR)])
    plsc.subcore_barrier()
    base = s * (_E_PAD // _NS)

    def run(y_ref):
        def chunk(k, carry):
            off = base + k * _CH
            pltpu.sync_copy(rowp.at[pl.ds(off, _CH)], idxr)
            pltpu.sync_copy(colp.at[pl.ds(off, _CH)], idxc)
            pltpu.async_copy(y_ref.at[idxr], rows, sem).wait()
            pltpu.sync_copy(rows, acc.at[idxc], add=True)
            return carry

        lax.fori_loop(0, _E_PAD // _NS // _CH, chunk, 0)

    @pl.when(c == 0)
    def _():
        run(ya)

    @pl.when(c == 1)
    def _():
        run(yb)

    plsc.subcore_barrier()
    pltpu.sync_copy(acc.at[pl.ds(s * _ZR, _ZR)],
                    out.at[pl.ds(c * _OFF + s * _ZR, _ZR)])


@functools.partial(
    pl.kernel,
    out_type=jax.ShapeDtypeStruct((_OFF + _ACC, 128), _f32),
    mesh=_mesh,
    scratch_types=[
        pltpu.VMEM((_CH,), _i32),
        pltpu.VMEM((_CH,), _i32),
        pltpu.VMEM((_CH, 128), _f32),
        pltpu.VMEM_SHARED((_ACC, 128), _f32),
        pltpu.SemaphoreType.DMA,
    ],
)
def _agg2_k(y2, rowp, colp, zer, qq, idxr, idxc, rows, acc, sem):
    c = lax.axis_index("c")
    s = lax.axis_index("s")
    pltpu.sync_copy(zer, acc.at[pl.ds(s * _ZR, _ZR)])
    plsc.subcore_barrier()
    base = c * (_E_PAD // 2) + s * (_E_PAD // (2 * _NS))

    def chunk(k, carry):
        off = base + k * _CH
        pltpu.sync_copy(rowp.at[pl.ds(off, _CH)], idxr)
        pltpu.sync_copy(colp.at[pl.ds(off, _CH)], idxc)
        pltpu.async_copy(y2.at[idxr], rows, sem).wait()
        pltpu.sync_copy(rows, acc.at[idxc], add=True)
        return carry

    lax.fori_loop(0, _E_PAD // (2 * _NS) // _CH, chunk, 0)
    plsc.subcore_barrier()
    pltpu.sync_copy(acc.at[pl.ds(s * _ZR, _ZR)],
                    qq.at[pl.ds(c * _OFF + s * _ZR, _ZR)])


def _edgeprep_body(row_ref, col_ref, colp_ref):
    rv = row_ref[...]
    cv = col_ref[...]
    colp_ref[...] = jnp.where(rv == cv, _DUMMY, cv)


def _pairnorm_body(x_ref, h_ref):
    xv = x_ref[...]
    m = jnp.mean(xv, axis=0, keepdims=True)
    xc = xv - m
    ms = jnp.sum(xc * xc) / _N
    h_ref[...] = xc / jnp.sqrt(1e-5 + ms)


def _scale_body(h_ref, pp_ref, ya_ref, yb_ref, dis_ref, invd_ref):
    pv = pp_ref[...]
    cnt = pv[:_N, 0:1] + pv[_OFF:_OFF + _N, 0:1]
    deg = cnt + 1.0
    dis = lax.rsqrt(deg)
    invd = 1.0 / deg
    y = h_ref[...] * dis
    ya_ref[...] = y[:, :128]
    yb_ref[...] = y[:, 128:]
    dis_ref[...] = dis
    invd_ref[...] = invd


def _mlp_body(a_ref, b_ref, h_ref, dis_ref, invd_ref, W1_ref, b1_ref, W2_ref,
              b2_ref, y2_ref, s2_ref):
    agg1 = jnp.concatenate([a_ref[...], b_ref[...]], axis=1)
    fa = dis_ref[...] * agg1 + invd_ref[...] * h_ref[...]
    t = jnp.dot(fa, W1_ref[...], preferred_element_type=_f32) + b1_ref[...]
    t = jnp.where(t > 0, t, 0.01 * t)
    xw2 = jnp.dot(t, W2_ref[...], preferred_element_type=_f32)
    y2_ref[...] = dis_ref[...] * xw2
    s2_ref[...] = invd_ref[...] * xw2 + b2_ref[...]


def _final_body(q0_ref, q1_ref, dis_ref, s2_ref, o_ref):
    o_ref[...] = dis_ref[...] * (q0_ref[...] + q1_ref[...]) + s2_ref[...]


_BR = 1000


def _mlp_call(oa, ob, h, dis, invd, W1, b1, W2, b2):
    grid = (_N // _BR,)
    return pl.pallas_call(
        _mlp_body,
        grid=grid,
        in_specs=[
            pl.BlockSpec((_BR, 128), lambda i: (i, 0)),
            pl.BlockSpec((_BR, 128), lambda i: (_OFF // _BR + i, 0)),
            pl.BlockSpec((_BR, _F), lambda i: (i, 0)),
            pl.BlockSpec((_BR, 1), lambda i: (i, 0)),
            pl.BlockSpec((_BR, 1), lambda i: (i, 0)),
            pl.BlockSpec((_F, _H), lambda i: (0, 0)),
            pl.BlockSpec((1, _H), lambda i: (0, 0)),
            pl.BlockSpec((_H, _C), lambda i: (0, 0)),
            pl.BlockSpec((1, _C), lambda i: (0, 0)),
        ],
        out_specs=[
            pl.BlockSpec((_BR, _C), lambda i: (i, 0)),
            pl.BlockSpec((_BR, _C), lambda i: (i, 0)),
        ],
        out_shape=[
            jax.ShapeDtypeStruct((_N, _C), _f32),
            jax.ShapeDtypeStruct((_N, _C), _f32),
        ],
    )(oa, ob, h, dis, invd, W1, b1, W2, b2)


def _final_call(q0, q1, dis, s2):
    grid = (_N // _BR,)
    return pl.pallas_call(
        _final_body,
        grid=grid,
        in_specs=[
            pl.BlockSpec((_BR, _C), lambda i: (i, 0)),
            pl.BlockSpec((_BR, _C), lambda i: (_OFF // _BR + i, 0)),
            pl.BlockSpec((_BR, 1), lambda i: (i, 0)),
            pl.BlockSpec((_BR, _C), lambda i: (i, 0)),
        ],
        out_specs=pl.BlockSpec((_BR, _C), lambda i: (i, 0)),
        out_shape=jax.ShapeDtypeStruct((_N, _C), _f32),
    )(q0, q1, dis, s2)


def kernel(x, edge_index, W1, b1, W2, b2):
    pad = jnp.zeros((_E_PAD - _E,), _i32)
    rowp2 = jnp.concatenate([edge_index[0], pad]).reshape(_E_PAD // 128, 128)
    colp2 = jnp.concatenate([edge_index[1], pad]).reshape(_E_PAD // 128, 128)

    colp2 = pl.pallas_call(
        _edgeprep_body,
        out_shape=jax.ShapeDtypeStruct((_E_PAD // 128, 128), _i32),
    )(rowp2, colp2)
    rowp_f = rowp2.reshape(-1)
    colp_f = colp2.reshape(-1)

    ones128 = jnp.ones((_CH, 128), _f32)
    zer128 = jnp.zeros((_ZR, 128), _f32)

    pp = _deg_k(colp_f, ones128, zer128)

    h = pl.pallas_call(
        _pairnorm_body,
        out_shape=jax.ShapeDtypeStruct((_N, _F), _f32),
    )(x)

    ya, yb, dis, invd = pl.pallas_call(
        _scale_body,
        out_shape=[
            jax.ShapeDtypeStruct((_N, 128), _f32),
            jax.ShapeDtypeStruct((_N, 128), _f32),
            jax.ShapeDtypeStruct((_N, 1), _f32),
            jax.ShapeDtypeStruct((_N, 1), _f32),
        ],
    )(h, pp)

    oo = _agg1_k(ya, yb, rowp_f, colp_f, zer128)

    y2, s2 = _mlp_call(oo, oo, h, dis, invd, W1, b1.reshape(1, _H), W2,
                       b2.reshape(1, _C))

    qq = _agg2_k(y2, rowp_f, colp_f, zer128)

    return _final_call(qq, qq, dis, s2)


# speedup vs baseline: 9.4347x; 1.2167x over previous
import functools

import jax
import jax.numpy as jnp
from jax import lax
from jax.experimental import pallas as pl
from jax.experimental.pallas import tpu as pltpu
from jax.experimental.pallas import tpu_sc as plsc

_N = 10000
_E = 160000
_F = 256
_H = 512
_C = 128

_NS = 16
_NC = 2
_CH = 128
_E_PAD = 163840
_ACC = 10240
_ZR = _ACC // _NS
_OFF = 11000
_DUMMY = _N

_f32 = jnp.float32
_i32 = jnp.int32

_mesh = plsc.VectorSubcoreMesh(core_axis_name="c", subcore_axis_name="s")


@functools.partial(
    pl.kernel,
    out_type=jax.ShapeDtypeStruct((_OFF + _ACC, 128), _f32),
    mesh=_mesh,
    scratch_types=[
        pltpu.VMEM((_CH, 128), _f32),
        pltpu.VMEM((_E_PAD // (2 * _NS) // _CH, _CH), _i32),
        pltpu.VMEM_SHARED((_ACC, 128), _f32),
    ],
)
def _deg_k(colp, ones_in, zer, pp, ones_v, idxc, acc):
    c = lax.axis_index("c")
    s = lax.axis_index("s")
    nch = _E_PAD // (2 * _NS) // _CH
    pltpu.sync_copy(zer, acc.at[pl.ds(s * _ZR, _ZR)])
    pltpu.sync_copy(ones_in, ones_v)
    pltpu.sync_copy(colp.at[pl.ds((c * _NS + s) * nch, nch)], idxc)
    plsc.subcore_barrier()

    def chunk(k, carry):
        pltpu.sync_copy(ones_v, acc.at[idxc.at[k]], add=True)
        return carry

    lax.fori_loop(0, nch, chunk, 0)
    plsc.subcore_barrier()
    pltpu.sync_copy(acc.at[pl.ds(s * _ZR, _ZR)],
                    pp.at[pl.ds(c * _OFF + s * _ZR, _ZR)])


@functools.partial(
    pl.kernel,
    out_type=jax.ShapeDtypeStruct((_OFF + _ACC, 128), _f32),
    mesh=_mesh,
    scratch_types=[
        pltpu.VMEM((_E_PAD // (2 * _NS) // _CH, _CH), _i32),
        pltpu.VMEM((_E_PAD // (2 * _NS) // _CH, _CH), _i32),
        pltpu.VMEM((_CH, 128), _f32),
        pltpu.VMEM((_CH, 128), _f32),
        pltpu.VMEM_SHARED((_ACC, 128), _f32),
        pltpu.SemaphoreType.DMA,
        pltpu.SemaphoreType.DMA,
    ],
)
def _agg1_k(ya, yb, rowp, colp, zer, out, idxr, idxc, r0, r1, acc, s0, s1):
    c = lax.axis_index("c")
    s = lax.axis_index("s")
    nch = _E_PAD // _NS // _CH
    nh = nch // 2
    pltpu.sync_copy(zer, acc.at[pl.ds(s * _ZR, _ZR)])
    plsc.subcore_barrier()

    def run(y_ref):
        for p in range(2):
            pltpu.sync_copy(rowp.at[pl.ds(s * nch + p * nh, nh)], idxr)
            pltpu.sync_copy(colp.at[pl.ds(s * nch + p * nh, nh)], idxc)
            pltpu.async_copy(y_ref.at[idxr.at[0]], r0, s0)
            pltpu.async_copy(y_ref.at[idxr.at[1]], r1, s1)

            def group(g, carry):
                k = 2 * g
                pltpu.make_async_copy(y_ref.at[idxr.at[k]], r0, s0).wait()
                pltpu.sync_copy(r0, acc.at[idxc.at[k]], add=True)
                pltpu.async_copy(y_ref.at[idxr.at[k + 2]], r0, s0)
                pltpu.make_async_copy(y_ref.at[idxr.at[k + 1]], r1, s1).wait()
                pltpu.sync_copy(r1, acc.at[idxc.at[k + 1]], add=True)
                pltpu.async_copy(y_ref.at[idxr.at[k + 3]], r1, s1)
                return carry

            lax.fori_loop(0, nh // 2 - 1, group, 0)
            pltpu.make_async_copy(y_ref.at[idxr.at[nh - 2]], r0, s0).wait()
            pltpu.sync_copy(r0, acc.at[idxc.at[nh - 2]], add=True)
            pltpu.make_async_copy(y_ref.at[idxr.at[nh - 1]], r1, s1).wait()
            pltpu.sync_copy(r1, acc.at[idxc.at[nh - 1]], add=True)

    @pl.when(c == 0)
    def _():
        run(ya)

    @pl.when(c == 1)
    def _():
        run(yb)

    plsc.subcore_barrier()
    pltpu.sync_copy(acc.at[pl.ds(s * _ZR, _ZR)],
                    out.at[pl.ds(c * _OFF + s * _ZR, _ZR)])


@functools.partial(
    pl.kernel,
    out_type=jax.ShapeDtypeStruct((_OFF + _ACC, 128), _f32),
    mesh=_mesh,
    scratch_types=[
        pltpu.VMEM((_E_PAD // (2 * _NS) // _CH, _CH), _i32),
        pltpu.VMEM((_E_PAD // (2 * _NS) // _CH, _CH), _i32),
        pltpu.VMEM((_CH, 128), _f32),
        pltpu.VMEM((_CH, 128), _f32),
        pltpu.VMEM_SHARED((_ACC, 128), _f32),
        pltpu.SemaphoreType.DMA,
        pltpu.SemaphoreType.DMA,
    ],
)
def _agg2_k(y2, rowp, colp, zer, qq, idxr, idxc, r0, r1, acc, s0, s1):
    c = lax.axis_index("c")
    s = lax.axis_index("s")
    nch = _E_PAD // (2 * _NS) // _CH
    pltpu.sync_copy(zer, acc.at[pl.ds(s * _ZR, _ZR)])
    tbase = (c * _NS + s) * nch
    pltpu.sync_copy(rowp.at[pl.ds(tbase, nch)], idxr)
    pltpu.sync_copy(colp.at[pl.ds(tbase, nch)], idxc)
    plsc.subcore_barrier()

    pltpu.async_copy(y2.at[idxr.at[0]], r0, s0)
    pltpu.async_copy(y2.at[idxr.at[1]], r1, s1)

    def group(g, carry):
        k = 2 * g
        pltpu.make_async_copy(y2.at[idxr.at[k]], r0, s0).wait()
        pltpu.sync_copy(r0, acc.at[idxc.at[k]], add=True)
        pltpu.async_copy(y2.at[idxr.at[k + 2]], r0, s0)
        pltpu.make_async_copy(y2.at[idxr.at[k + 1]], r1, s1).wait()
        pltpu.sync_copy(r1, acc.at[idxc.at[k + 1]], add=True)
        pltpu.async_copy(y2.at[idxr.at[k + 3]], r1, s1)
        return carry

    lax.fori_loop(0, nch // 2 - 1, group, 0)
    pltpu.make_async_copy(y2.at[idxr.at[nch - 2]], r0, s0).wait()
    pltpu.sync_copy(r0, acc.at[idxc.at[nch - 2]], add=True)
    pltpu.make_async_copy(y2.at[idxr.at[nch - 1]], r1, s1).wait()
    pltpu.sync_copy(r1, acc.at[idxc.at[nch - 1]], add=True)

    plsc.subcore_barrier()
    pltpu.sync_copy(acc.at[pl.ds(s * _ZR, _ZR)],
                    qq.at[pl.ds(c * _OFF + s * _ZR, _ZR)])


def _edgeprep_body(row_ref, col_ref, colp_ref):
    rv = row_ref[...]
    cv = col_ref[...]
    colp_ref[...] = jnp.where(rv == cv, _DUMMY, cv)


def _pairnorm_body(x_ref, h_ref):
    xv = x_ref[...]
    m = jnp.mean(xv, axis=0, keepdims=True)
    xc = xv - m
    ms = jnp.sum(xc * xc) / _N
    h_ref[...] = xc / jnp.sqrt(1e-5 + ms)


def _scale_body(h_ref, pp_ref, ya_ref, yb_ref, dis_ref, invd_ref):
    pv = pp_ref[...]
    cnt = pv[:_N, 0:1] + pv[_OFF:_OFF + _N, 0:1]
    deg = cnt + 1.0
    dis = lax.rsqrt(deg)
    invd = 1.0 / deg
    y = h_ref[...] * dis
    ya_ref[...] = y[:, :128]
    yb_ref[...] = y[:, 128:]
    dis_ref[...] = dis
    invd_ref[...] = invd


def _mlp_body(a_ref, b_ref, h_ref, dis_ref, invd_ref, W1_ref, b1_ref, W2_ref,
              b2_ref, y2_ref, s2_ref):
    agg1 = jnp.concatenate([a_ref[...], b_ref[...]], axis=1)
    fa = dis_ref[...] * agg1 + invd_ref[...] * h_ref[...]
    t = jnp.dot(fa, W1_ref[...], preferred_element_type=_f32) + b1_ref[...]
    t = jnp.where(t > 0, t, 0.01 * t)
    xw2 = jnp.dot(t, W2_ref[...], preferred_element_type=_f32)
    y2_ref[...] = dis_ref[...] * xw2
    s2_ref[...] = invd_ref[...] * xw2 + b2_ref[...]


def _final_body(q0_ref, q1_ref, dis_ref, s2_ref, o_ref):
    o_ref[...] = dis_ref[...] * (q0_ref[...] + q1_ref[...]) + s2_ref[...]


_BR = 1000


def _mlp_call(oa, ob, h, dis, invd, W1, b1, W2, b2):
    grid = (_N // _BR,)
    return pl.pallas_call(
        _mlp_body,
        grid=grid,
        in_specs=[
            pl.BlockSpec((_BR, 128), lambda i: (i, 0)),
            pl.BlockSpec((_BR, 128), lambda i: (_OFF // _BR + i, 0)),
            pl.BlockSpec((_BR, _F), lambda i: (i, 0)),
            pl.BlockSpec((_BR, 1), lambda i: (i, 0)),
            pl.BlockSpec((_BR, 1), lambda i: (i, 0)),
            pl.BlockSpec((_F, _H), lambda i: (0, 0)),
            pl.BlockSpec((1, _H), lambda i: (0, 0)),
            pl.BlockSpec((_H, _C), lambda i: (0, 0)),
            pl.BlockSpec((1, _C), lambda i: (0, 0)),
        ],
        out_specs=[
            pl.BlockSpec((_BR, _C), lambda i: (i, 0)),
            pl.BlockSpec((_BR, _C), lambda i: (i, 0)),
        ],
        out_shape=[
            jax.ShapeDtypeStruct((_N, _C), _f32),
            jax.ShapeDtypeStruct((_N, _C), _f32),
        ],
    )(oa, ob, h, dis, invd, W1, b1, W2, b2)


def _final_call(q0, q1, dis, s2):
    grid = (_N // _BR,)
    return pl.pallas_call(
        _final_body,
        grid=grid,
        in_specs=[
            pl.BlockSpec((_BR, _C), lambda i: (i, 0)),
            pl.BlockSpec((_BR, _C), lambda i: (_OFF // _BR + i, 0)),
            pl.BlockSpec((_BR, 1), lambda i: (i, 0)),
            pl.BlockSpec((_BR, _C), lambda i: (i, 0)),
        ],
        out_specs=pl.BlockSpec((_BR, _C), lambda i: (i, 0)),
        out_shape=jax.ShapeDtypeStruct((_N, _C), _f32),
    )(q0, q1, dis, s2)


def kernel(x, edge_index, W1, b1, W2, b2):
    pad = jnp.zeros((_E_PAD - _E,), _i32)
    rowp2 = jnp.concatenate([edge_index[0], pad]).reshape(_E_PAD // 128, 128)
    colp2 = jnp.concatenate([edge_index[1], pad]).reshape(_E_PAD // 128, 128)

    colp2 = pl.pallas_call(
        _edgeprep_body,
        out_shape=jax.ShapeDtypeStruct((_E_PAD // 128, 128), _i32),
    )(rowp2, colp2)

    ones128 = jnp.ones((_CH, 128), _f32)
    zer128 = jnp.zeros((_ZR, 128), _f32)

    pp = _deg_k(colp2, ones128, zer128)

    h = pl.pallas_call(
        _pairnorm_body,
        out_shape=jax.ShapeDtypeStruct((_N, _F), _f32),
    )(x)

    ya, yb, dis, invd = pl.pallas_call(
        _scale_body,
        out_shape=[
            jax.ShapeDtypeStruct((_N, 128), _f32),
            jax.ShapeDtypeStruct((_N, 128), _f32),
            jax.ShapeDtypeStruct((_N, 1), _f32),
            jax.ShapeDtypeStruct((_N, 1), _f32),
        ],
    )(h, pp)

    oo = _agg1_k(ya, yb, rowp2, colp2, zer128)

    y2, s2 = _mlp_call(oo, oo, h, dis, invd, W1, b1.reshape(1, _H), W2,
                       b2.reshape(1, _C))

    qq = _agg2_k(y2, rowp2, colp2, zer128)

    return _final_call(qq, qq, dis, s2)


# speedup vs baseline: 22.6145x; 2.3970x over previous
import functools

import jax
import jax.numpy as jnp
from jax import lax
from jax.experimental import pallas as pl
from jax.experimental.pallas import tpu as pltpu
from jax.experimental.pallas import tpu_sc as plsc

_N = 10000
_E = 160000
_F = 256
_H = 512
_C = 128

_NS = 16
_NC = 2
_CH = 128
_E_PAD = 163840
_ACC = 10240
_ZR = _ACC // _NS
_OFF = 11000
_DUMMY = _N

_f32 = jnp.float32
_i32 = jnp.int32

_mesh = plsc.VectorSubcoreMesh(core_axis_name="c", subcore_axis_name="s")


@functools.partial(
    pl.kernel,
    out_type=jax.ShapeDtypeStruct((_OFF + _ACC, 128), _f32),
    mesh=_mesh,
    scratch_types=[
        pltpu.VMEM((_CH, 128), _f32),
        pltpu.VMEM((_E_PAD // (2 * _NS) // _CH, _CH), _i32),
        pltpu.VMEM_SHARED((_ACC, 128), _f32),
    ],
)
def _deg_k(colp, ones_in, zer, pp, ones_v, idxc, acc):
    c = lax.axis_index("c")
    s = lax.axis_index("s")
    nch = _E_PAD // (2 * _NS) // _CH
    pltpu.sync_copy(zer, acc.at[pl.ds(s * _ZR, _ZR)])
    pltpu.sync_copy(ones_in, ones_v)
    pltpu.sync_copy(colp.at[pl.ds((c * _NS + s) * nch, nch)], idxc)
    plsc.subcore_barrier()

    def chunk(k, carry):
        pltpu.sync_copy(ones_v, acc.at[idxc.at[k]], add=True)
        return carry

    lax.fori_loop(0, nch, chunk, 0)
    plsc.subcore_barrier()
    pltpu.sync_copy(acc.at[pl.ds(s * _ZR, _ZR)],
                    pp.at[pl.ds(c * _OFF + s * _ZR, _ZR)])


@functools.partial(
    pl.kernel,
    out_type=jax.ShapeDtypeStruct((_OFF + _ACC, 128), _f32),
    mesh=_mesh,
    scratch_types=[
        pltpu.VMEM((_E_PAD // (2 * _NS) // _CH, _CH), _i32),
        pltpu.VMEM((_E_PAD // (2 * _NS) // _CH, _CH), _i32),
        pltpu.VMEM((_CH, 128), _f32),
        pltpu.VMEM((_CH, 128), _f32),
        pltpu.VMEM_SHARED((_ACC, 128), _f32),
        pltpu.SemaphoreType.DMA,
        pltpu.SemaphoreType.DMA,
    ],
)
def _agg1_k(ya, yb, rowp, colp, zer, out, idxr, idxc, r0, r1, acc, s0, s1):
    c = lax.axis_index("c")
    s = lax.axis_index("s")
    nch = _E_PAD // _NS // _CH
    nh = nch // 2
    pltpu.sync_copy(zer, acc.at[pl.ds(s * _ZR, _ZR)])
    plsc.subcore_barrier()

    def run(y_ref):
        for p in range(2):
            pltpu.sync_copy(rowp.at[pl.ds(s * nch + p * nh, nh)], idxr)
            pltpu.sync_copy(colp.at[pl.ds(s * nch + p * nh, nh)], idxc)
            pltpu.async_copy(y_ref.at[idxr.at[0]], r0, s0)
            pltpu.async_copy(y_ref.at[idxr.at[1]], r1, s1)

            def group(g, carry):
                k = 2 * g
                pltpu.make_async_copy(y_ref.at[idxr.at[k]], r0, s0).wait()
                pltpu.sync_copy(r0, acc.at[idxc.at[k]], add=True)
                pltpu.async_copy(y_ref.at[idxr.at[k + 2]], r0, s0)
                pltpu.make_async_copy(y_ref.at[idxr.at[k + 1]], r1, s1).wait()
                pltpu.sync_copy(r1, acc.at[idxc.at[k + 1]], add=True)
                pltpu.async_copy(y_ref.at[idxr.at[k + 3]], r1, s1)
                return carry

            lax.fori_loop(0, nh // 2 - 1, group, 0)
            pltpu.make_async_copy(y_ref.at[idxr.at[nh - 2]], r0, s0).wait()
            pltpu.sync_copy(r0, acc.at[idxc.at[nh - 2]], add=True)
            pltpu.make_async_copy(y_ref.at[idxr.at[nh - 1]], r1, s1).wait()
            pltpu.sync_copy(r1, acc.at[idxc.at[nh - 1]], add=True)

    @pl.when(c == 0)
    def _():
        run(ya)

    @pl.when(c == 1)
    def _():
        run(yb)

    plsc.subcore_barrier()
    pltpu.sync_copy(acc.at[pl.ds(s * _ZR, _ZR)],
                    out.at[pl.ds(c * _OFF + s * _ZR, _ZR)])


@functools.partial(
    pl.kernel,
    out_type=jax.ShapeDtypeStruct((_OFF + _ACC, 128), _f32),
    mesh=_mesh,
    scratch_types=[
        pltpu.VMEM((_E_PAD // (2 * _NS) // _CH, _CH), _i32),
        pltpu.VMEM((_E_PAD // (2 * _NS) // _CH, _CH), _i32),
        pltpu.VMEM((_CH, 128), _f32),
        pltpu.VMEM((_CH, 128), _f32),
        pltpu.VMEM_SHARED((_ACC, 128), _f32),
        pltpu.SemaphoreType.DMA,
        pltpu.SemaphoreType.DMA,
    ],
)
def _agg2_k(y2, rowp, colp, zer, qq, idxr, idxc, r0, r1, acc, s0, s1):
    c = lax.axis_index("c")
    s = lax.axis_index("s")
    nch = _E_PAD // (2 * _NS) // _CH
    pltpu.sync_copy(zer, acc.at[pl.ds(s * _ZR, _ZR)])
    tbase = (c * _NS + s) * nch
    pltpu.sync_copy(rowp.at[pl.ds(tbase, nch)], idxr)
    pltpu.sync_copy(colp.at[pl.ds(tbase, nch)], idxc)
    plsc.subcore_barrier()

    pltpu.async_copy(y2.at[idxr.at[0]], r0, s0)
    pltpu.async_copy(y2.at[idxr.at[1]], r1, s1)

    def group(g, carry):
        k = 2 * g
        pltpu.make_async_copy(y2.at[idxr.at[k]], r0, s0).wait()
        pltpu.sync_copy(r0, acc.at[idxc.at[k]], add=True)
        pltpu.async_copy(y2.at[idxr.at[k + 2]], r0, s0)
        pltpu.make_async_copy(y2.at[idxr.at[k + 1]], r1, s1).wait()
        pltpu.sync_copy(r1, acc.at[idxc.at[k + 1]], add=True)
        pltpu.async_copy(y2.at[idxr.at[k + 3]], r1, s1)
        return carry

    lax.fori_loop(0, nch // 2 - 1, group, 0)
    pltpu.make_async_copy(y2.at[idxr.at[nch - 2]], r0, s0).wait()
    pltpu.sync_copy(r0, acc.at[idxc.at[nch - 2]], add=True)
    pltpu.make_async_copy(y2.at[idxr.at[nch - 1]], r1, s1).wait()
    pltpu.sync_copy(r1, acc.at[idxc.at[nch - 1]], add=True)

    plsc.subcore_barrier()
    pltpu.sync_copy(acc.at[pl.ds(s * _ZR, _ZR)],
                    qq.at[pl.ds(c * _OFF + s * _ZR, _ZR)])


def _edgeprep_body(row_ref, col_ref, colp_ref):
    rv = row_ref[...]
    cv = col_ref[...]
    colp_ref[...] = jnp.where(rv == cv, _DUMMY, cv)


def _pairnorm_body(x_ref, h_ref):
    xv = x_ref[...]
    m = jnp.mean(xv, axis=0, keepdims=True)
    xc = xv - m
    ms = jnp.sum(xc * xc) / _N
    h_ref[...] = xc / jnp.sqrt(1e-5 + ms)


def _scale_body(h_ref, pp_ref, ya_ref, yb_ref, dis_ref, invd_ref):
    pv = pp_ref[...]
    cnt = pv[:_N, 0:1] + pv[_OFF:_OFF + _N, 0:1]
    deg = cnt + 1.0
    dis = lax.rsqrt(deg)
    invd = 1.0 / deg
    y = h_ref[...] * dis
    ya_ref[...] = y[:, :128]
    yb_ref[...] = y[:, 128:]
    dis_ref[...] = dis
    invd_ref[...] = invd


def _mlp_body(a_ref, b_ref, h_ref, dis_ref, invd_ref, W1_ref, b1_ref, W2_ref,
              b2_ref, y2_ref, s2_ref):
    agg1 = jnp.concatenate([a_ref[...], b_ref[...]], axis=1)
    fa = dis_ref[...] * agg1 + invd_ref[...] * h_ref[...]
    t = jnp.dot(fa, W1_ref[...], preferred_element_type=_f32) + b1_ref[...]
    t = jnp.where(t > 0, t, 0.01 * t)
    xw2 = jnp.dot(t, W2_ref[...], preferred_element_type=_f32)
    y2_ref[...] = dis_ref[...] * xw2
    s2_ref[...] = invd_ref[...] * xw2 + b2_ref[...]


def _final_body(q0_ref, q1_ref, dis_ref, s2_ref, o_ref):
    o_ref[...] = dis_ref[...] * (q0_ref[...] + q1_ref[...]) + s2_ref[...]


_BR = 1000


def _mlp_call(oa, ob, h, dis, invd, W1, b1, W2, b2):
    grid = (_N // _BR,)
    return pl.pallas_call(
        _mlp_body,
        grid=grid,
        in_specs=[
            pl.BlockSpec((_BR, 128), lambda i: (i, 0)),
            pl.BlockSpec((_BR, 128), lambda i: (_OFF // _BR + i, 0)),
            pl.BlockSpec((_BR, _F), lambda i: (i, 0)),
            pl.BlockSpec((_BR, 1), lambda i: (i, 0)),
            pl.BlockSpec((_BR, 1), lambda i: (i, 0)),
            pl.BlockSpec((_F, _H), lambda i: (0, 0)),
            pl.BlockSpec((1, _H), lambda i: (0, 0)),
            pl.BlockSpec((_H, _C), lambda i: (0, 0)),
            pl.BlockSpec((1, _C), lambda i: (0, 0)),
        ],
        out_specs=[
            pl.BlockSpec((_BR, _C), lambda i: (i, 0)),
            pl.BlockSpec((_BR, _C), lambda i: (i, 0)),
        ],
        out_shape=[
            jax.ShapeDtypeStruct((_N, _C), _f32),
            jax.ShapeDtypeStruct((_N, _C), _f32),
        ],
    )(oa, ob, h, dis, invd, W1, b1, W2, b2)


def _final_call(q0, q1, dis, s2):
    grid = (_N // _BR,)
    return pl.pallas_call(
        _final_body,
        grid=grid,
        in_specs=[
            pl.BlockSpec((_BR, _C), lambda i: (i, 0)),
            pl.BlockSpec((_BR, _C), lambda i: (_OFF // _BR + i, 0)),
            pl.BlockSpec((_BR, 1), lambda i: (i, 0)),
            pl.BlockSpec((_BR, _C), lambda i: (i, 0)),
        ],
        out_specs=pl.BlockSpec((_BR, _C), lambda i: (i, 0)),
        out_shape=jax.ShapeDtypeStruct((_N, _C), _f32),
    )(q0, q1, dis, s2)


def kernel(x, edge_index, W1, b1, W2, b2):
    npad = _E_PAD - _E
    j = jnp.arange(npad, dtype=_i32)
    pad_r = j % _N
    pad_c = _N + 1 + j % (_ACC - _N - 1)
    rowp2 = jnp.concatenate([edge_index[0], pad_r]).reshape(_E_PAD // 128, 128)
    colp2 = jnp.concatenate([edge_index[1], pad_c]).reshape(_E_PAD // 128, 128)

    colp2 = pl.pallas_call(
        _edgeprep_body,
        out_shape=jax.ShapeDtypeStruct((_E_PAD // 128, 128), _i32),
    )(rowp2, colp2)

    ones128 = jnp.ones((_CH, 128), _f32)
    zer128 = jnp.zeros((_ZR, 128), _f32)

    pp = _deg_k(colp2, ones128, zer128)

    h = pl.pallas_call(
        _pairnorm_body,
        out_shape=jax.ShapeDtypeStruct((_N, _F), _f32),
    )(x)

    ya, yb, dis, invd = pl.pallas_call(
        _scale_body,
        out_shape=[
            jax.ShapeDtypeStruct((_N, 128), _f32),
            jax.ShapeDtypeStruct((_N, 128), _f32),
            jax.ShapeDtypeStruct((_N, 1), _f32),
            jax.ShapeDtypeStruct((_N, 1), _f32),
        ],
    )(h, pp)

    oo = _agg1_k(ya, yb, rowp2, colp2, zer128)

    y2, s2 = _mlp_call(oo, oo, h, dis, invd, W1, b1.reshape(1, _H), W2,
                       b2.reshape(1, _C))

    qq = _agg2_k(y2, rowp2, colp2, zer128)

    return _final_call(qq, qq, dis, s2)


# speedup vs baseline: 24.1294x; 1.0670x over previous
import functools

import jax
import jax.numpy as jnp
from jax import lax
from jax.experimental import pallas as pl
from jax.experimental.pallas import tpu as pltpu
from jax.experimental.pallas import tpu_sc as plsc

_N = 10000
_E = 160000
_F = 256
_H = 512
_C = 128

_NS = 16
_NC = 2
_CH = 128
_E_PAD = 163840
_ACC = 10240
_ZR = _ACC // _NS
_OFF = 11000
_DUMMY = _N

_f32 = jnp.float32
_i32 = jnp.int32

_mesh = plsc.VectorSubcoreMesh(core_axis_name="c", subcore_axis_name="s")


@functools.partial(
    pl.kernel,
    out_type=jax.ShapeDtypeStruct((_OFF + _ACC, 32), _f32),
    mesh=_mesh,
    scratch_types=[
        pltpu.VMEM((_CH, 32), _f32),
        pltpu.VMEM((_E_PAD // (2 * _NS) // _CH, _CH), _i32),
        pltpu.VMEM_SHARED((_ACC, 32), _f32),
    ],
)
def _deg_k(colp, ones_in, zer, pp, ones_v, idxc, acc):
    c = lax.axis_index("c")
    s = lax.axis_index("s")
    nch = _E_PAD // (2 * _NS) // _CH
    pltpu.sync_copy(zer, acc.at[pl.ds(s * _ZR, _ZR)])
    pltpu.sync_copy(ones_in, ones_v)
    pltpu.sync_copy(colp.at[pl.ds((c * _NS + s) * nch, nch)], idxc)
    plsc.subcore_barrier()

    def chunk(k, carry):
        pltpu.sync_copy(ones_v, acc.at[idxc.at[k]], add=True)
        return carry

    lax.fori_loop(0, nch, chunk, 0)
    plsc.subcore_barrier()
    pltpu.sync_copy(acc.at[pl.ds(s * _ZR, _ZR)],
                    pp.at[pl.ds(c * _OFF + s * _ZR, _ZR)])


@functools.partial(
    pl.kernel,
    out_type=jax.ShapeDtypeStruct((_OFF + _ACC, 128), _f32),
    mesh=_mesh,
    scratch_types=[
        pltpu.VMEM((_E_PAD // (2 * _NS) // _CH, _CH), _i32),
        pltpu.VMEM((_E_PAD // (2 * _NS) // _CH, _CH), _i32),
        pltpu.VMEM((_CH, 128), _f32),
        pltpu.VMEM((_CH, 128), _f32),
        pltpu.VMEM_SHARED((_ACC, 128), _f32),
        pltpu.SemaphoreType.DMA,
        pltpu.SemaphoreType.DMA,
    ],
)
def _agg1_k(ya, yb, rowp, colp, zer, out, idxr, idxc, r0, r1, acc, s0, s1):
    c = lax.axis_index("c")
    s = lax.axis_index("s")
    nch = _E_PAD // _NS // _CH
    nh = nch // 2
    pltpu.sync_copy(zer, acc.at[pl.ds(s * _ZR, _ZR)])
    plsc.subcore_barrier()

    def run(y_ref):
        for p in range(2):
            pltpu.sync_copy(rowp.at[pl.ds(s * nch + p * nh, nh)], idxr)
            pltpu.sync_copy(colp.at[pl.ds(s * nch + p * nh, nh)], idxc)
            pltpu.async_copy(y_ref.at[idxr.at[0]], r0, s0)
            pltpu.async_copy(y_ref.at[idxr.at[1]], r1, s1)

            def group(g, carry):
                k = 2 * g
                pltpu.make_async_copy(y_ref.at[idxr.at[k]], r0, s0).wait()
                pltpu.sync_copy(r0, acc.at[idxc.at[k]], add=True)
                pltpu.async_copy(y_ref.at[idxr.at[k + 2]], r0, s0)
                pltpu.make_async_copy(y_ref.at[idxr.at[k + 1]], r1, s1).wait()
                pltpu.sync_copy(r1, acc.at[idxc.at[k + 1]], add=True)
                pltpu.async_copy(y_ref.at[idxr.at[k + 3]], r1, s1)
                return carry

            lax.fori_loop(0, nh // 2 - 1, group, 0)
            pltpu.make_async_copy(y_ref.at[idxr.at[nh - 2]], r0, s0).wait()
            pltpu.sync_copy(r0, acc.at[idxc.at[nh - 2]], add=True)
            pltpu.make_async_copy(y_ref.at[idxr.at[nh - 1]], r1, s1).wait()
            pltpu.sync_copy(r1, acc.at[idxc.at[nh - 1]], add=True)

    @pl.when(c == 0)
    def _():
        run(ya)

    @pl.when(c == 1)
    def _():
        run(yb)

    plsc.subcore_barrier()
    pltpu.sync_copy(acc.at[pl.ds(s * _ZR, _ZR)],
                    out.at[pl.ds(c * _OFF + s * _ZR, _ZR)])


@functools.partial(
    pl.kernel,
    out_type=jax.ShapeDtypeStruct((_OFF + _ACC, 128), _f32),
    mesh=_mesh,
    scratch_types=[
        pltpu.VMEM((_E_PAD // (2 * _NS) // _CH, _CH), _i32),
        pltpu.VMEM((_E_PAD // (2 * _NS) // _CH, _CH), _i32),
        pltpu.VMEM((_CH, 128), _f32),
        pltpu.VMEM((_CH, 128), _f32),
        pltpu.VMEM_SHARED((_ACC, 128), _f32),
        pltpu.SemaphoreType.DMA,
        pltpu.SemaphoreType.DMA,
    ],
)
def _agg2_k(y2, rowp, colp, zer, qq, idxr, idxc, r0, r1, acc, s0, s1):
    c = lax.axis_index("c")
    s = lax.axis_index("s")
    nch = _E_PAD // (2 * _NS) // _CH
    pltpu.sync_copy(zer, acc.at[pl.ds(s * _ZR, _ZR)])
    tbase = (c * _NS + s) * nch
    pltpu.sync_copy(rowp.at[pl.ds(tbase, nch)], idxr)
    pltpu.sync_copy(colp.at[pl.ds(tbase, nch)], idxc)
    plsc.subcore_barrier()

    pltpu.async_copy(y2.at[idxr.at[0]], r0, s0)
    pltpu.async_copy(y2.at[idxr.at[1]], r1, s1)

    def group(g, carry):
        k = 2 * g
        pltpu.make_async_copy(y2.at[idxr.at[k]], r0, s0).wait()
        pltpu.sync_copy(r0, acc.at[idxc.at[k]], add=True)
        pltpu.async_copy(y2.at[idxr.at[k + 2]], r0, s0)
        pltpu.make_async_copy(y2.at[idxr.at[k + 1]], r1, s1).wait()
        pltpu.sync_copy(r1, acc.at[idxc.at[k + 1]], add=True)
        pltpu.async_copy(y2.at[idxr.at[k + 3]], r1, s1)
        return carry

    lax.fori_loop(0, nch // 2 - 1, group, 0)
    pltpu.make_async_copy(y2.at[idxr.at[nch - 2]], r0, s0).wait()
    pltpu.sync_copy(r0, acc.at[idxc.at[nch - 2]], add=True)
    pltpu.make_async_copy(y2.at[idxr.at[nch - 1]], r1, s1).wait()
    pltpu.sync_copy(r1, acc.at[idxc.at[nch - 1]], add=True)

    plsc.subcore_barrier()
    pltpu.sync_copy(acc.at[pl.ds(s * _ZR, _ZR)],
                    qq.at[pl.ds(c * _OFF + s * _ZR, _ZR)])


def _edgeprep_body(row_ref, col_ref, rowp_ref, colp_ref):
    rv = row_ref[...]
    cv = col_ref[...]
    cp = jnp.where(rv == cv, _DUMMY, cv)
    nprow = (_E_PAD - _E) // 128
    j = (lax.broadcasted_iota(_i32, (nprow, 128), 0) * 128 +
         lax.broadcasted_iota(_i32, (nprow, 128), 1))
    rowp_ref[...] = jnp.concatenate([rv, j % _N], axis=0)
    colp_ref[...] = jnp.concatenate([cp, _N + 1 + j % (_ACC - _N - 1)], axis=0)


def _pairnorm_body(x_ref, h_ref):
    xv = x_ref[...]
    m = jnp.mean(xv, axis=0, keepdims=True)
    xc = xv - m
    ms = jnp.sum(xc * xc) / _N
    h_ref[...] = xc / jnp.sqrt(1e-5 + ms)


def _scale_body(h_ref, pp_ref, ya_ref, yb_ref, dis_ref, invd_ref):
    pv = pp_ref[...]
    cnt = pv[:_N, 0:1] + pv[_OFF:_OFF + _N, 0:1]
    deg = cnt + 1.0
    dis = lax.rsqrt(deg)
    invd = 1.0 / deg
    y = h_ref[...] * dis
    ya_ref[...] = y[:, :128]
    yb_ref[...] = y[:, 128:]
    dis_ref[...] = dis
    invd_ref[...] = invd


def _mlp_body(a_ref, b_ref, h_ref, dis_ref, invd_ref, W1_ref, b1_ref, W2_ref,
              b2_ref, y2_ref, s2_ref):
    agg1 = jnp.concatenate([a_ref[...], b_ref[...]], axis=1)
    fa = dis_ref[...] * agg1 + invd_ref[...] * h_ref[...]
    t = jnp.dot(fa, W1_ref[...], preferred_element_type=_f32) + b1_ref[...]
    t = jnp.where(t > 0, t, 0.01 * t)
    xw2 = jnp.dot(t, W2_ref[...], preferred_element_type=_f32)
    y2_ref[...] = dis_ref[...] * xw2
    s2_ref[...] = invd_ref[...] * xw2 + b2_ref[...]


def _final_body(q0_ref, q1_ref, dis_ref, s2_ref, o_ref):
    o_ref[...] = dis_ref[...] * (q0_ref[...] + q1_ref[...]) + s2_ref[...]


_BR = 1000


def _mlp_call(oa, ob, h, dis, invd, W1, b1, W2, b2):
    grid = (_N // _BR,)
    return pl.pallas_call(
        _mlp_body,
        grid=grid,
        in_specs=[
            pl.BlockSpec((_BR, 128), lambda i: (i, 0)),
            pl.BlockSpec((_BR, 128), lambda i: (_OFF // _BR + i, 0)),
            pl.BlockSpec((_BR, _F), lambda i: (i, 0)),
            pl.BlockSpec((_BR, 1), lambda i: (i, 0)),
            pl.BlockSpec((_BR, 1), lambda i: (i, 0)),
            pl.BlockSpec((_F, _H), lambda i: (0, 0)),
            pl.BlockSpec((1, _H), lambda i: (0, 0)),
            pl.BlockSpec((_H, _C), lambda i: (0, 0)),
            pl.BlockSpec((1, _C), lambda i: (0, 0)),
        ],
        out_specs=[
            pl.BlockSpec((_BR, _C), lambda i: (i, 0)),
            pl.BlockSpec((_BR, _C), lambda i: (i, 0)),
        ],
        out_shape=[
            jax.ShapeDtypeStruct((_N, _C), _f32),
            jax.ShapeDtypeStruct((_N, _C), _f32),
        ],
    )(oa, ob, h, dis, invd, W1, b1, W2, b2)


def _final_call(q0, q1, dis, s2):
    grid = (_N // _BR,)
    return pl.pallas_call(
        _final_body,
        grid=grid,
        in_specs=[
            pl.BlockSpec((_BR, _C), lambda i: (i, 0)),
            pl.BlockSpec((_BR, _C), lambda i: (_OFF // _BR + i, 0)),
            pl.BlockSpec((_BR, 1), lambda i: (i, 0)),
            pl.BlockSpec((_BR, _C), lambda i: (i, 0)),
        ],
        out_specs=pl.BlockSpec((_BR, _C), lambda i: (i, 0)),
        out_shape=jax.ShapeDtypeStruct((_N, _C), _f32),
    )(q0, q1, dis, s2)


def kernel(x, edge_index, W1, b1, W2, b2):
    row2 = edge_index[0].reshape(_E // 128, 128)
    col2 = edge_index[1].reshape(_E // 128, 128)
    rowp2, colp2 = pl.pallas_call(
        _edgeprep_body,
        out_shape=[
            jax.ShapeDtypeStruct((_E_PAD // 128, 128), _i32),
            jax.ShapeDtypeStruct((_E_PAD // 128, 128), _i32),
        ],
    )(row2, col2)

    ones32 = jnp.ones((_CH, 32), _f32)
    zer32 = jnp.zeros((_ZR, 32), _f32)
    zer128 = jnp.zeros((_ZR, 128), _f32)

    pp = _deg_k(colp2, ones32, zer32)

    h = pl.pallas_call(
        _pairnorm_body,
        out_shape=jax.ShapeDtypeStruct((_N, _F), _f32),
    )(x)

    ya, yb, dis, invd = pl.pallas_call(
        _scale_body,
        out_shape=[
            jax.ShapeDtypeStruct((_N, 128), _f32),
            jax.ShapeDtypeStruct((_N, 128), _f32),
            jax.ShapeDtypeStruct((_N, 1), _f32),
            jax.ShapeDtypeStruct((_N, 1), _f32),
        ],
    )(h, pp)

    oo = _agg1_k(ya, yb, rowp2, colp2, zer128)

    y2, s2 = _mlp_call(oo, oo, h, dis, invd, W1, b1.reshape(1, _H), W2,
                       b2.reshape(1, _C))

    qq = _agg2_k(y2, rowp2, colp2, zer128)

    return _final_call(qq, qq, dis, s2)
